# baseline (speedup 1.0000x reference)
import functools

import jax
import jax.numpy as jnp
from jax import lax
from jax.experimental import pallas as pl
from jax.experimental.pallas import tpu as pltpu
from jax.experimental.pallas import tpu_sc as plsc

_MU = 3.5
_LANES = 16
_NCHUNK = 2


@jax.jit
def kernel(user, item, user_biases, item_biases):
    batch = user.shape[0]
    info = plsc.get_sparse_core_info()
    num_subcores = info.num_subcores
    num_cores = info.num_cores
    num_workers = num_cores * num_subcores
    b_per_w = batch // num_workers
    chunk = b_per_w // _NCHUNK

    mesh = plsc.VectorSubcoreMesh(core_axis_name="c", subcore_axis_name="s",
                                  num_cores=num_cores)

    idx_t = pltpu.VMEM((chunk,), jnp.int32)
    val_t = pltpu.VMEM((chunk,), jnp.float32)

    @functools.partial(
        pl.kernel,
        mesh=mesh,
        out_type=jax.ShapeDtypeStruct((batch,), jnp.float32),
        scratch_types=[
            idx_t, idx_t, idx_t, idx_t,
            val_t, val_t, val_t, val_t,
            val_t, val_t,
            pltpu.SemaphoreType.DMA, pltpu.SemaphoreType.DMA,
            pltpu.SemaphoreType.DMA, pltpu.SemaphoreType.DMA,
        ],
    )
    def sc_kernel(user_hbm, item_hbm, ub_hbm, ib_hbm, out_hbm,
                  uidx0, iidx0, uidx1, iidx1,
                  bu0, bi0, bu1, bi1, out0, out1,
                  sem_a, sem_b, sem_c, sem_d):
        wid = lax.axis_index("s") * num_cores + lax.axis_index("c")
        base = wid * b_per_w
        sl0 = pl.ds(base, chunk)
        sl1 = pl.ds(base + chunk, chunk)
        cp_u0 = pltpu.async_copy(user_hbm.at[sl0], uidx0, sem_a)
        cp_i0 = pltpu.async_copy(item_hbm.at[sl0], iidx0, sem_b)
        cp_u1 = pltpu.async_copy(user_hbm.at[sl1], uidx1, sem_c)
        cp_i1 = pltpu.async_copy(item_hbm.at[sl1], iidx1, sem_d)
        cp_u0.wait()
        g_u0 = pltpu.async_copy(ub_hbm.at[uidx0], bu0, sem_a)
        cp_i0.wait()
        g_i0 = pltpu.async_copy(ib_hbm.at[iidx0], bi0, sem_b)
        cp_u1.wait()
        g_u1 = pltpu.async_copy(ub_hbm.at[uidx1], bu1, sem_c)
        cp_i1.wait()
        g_i1 = pltpu.async_copy(ib_hbm.at[iidx1], bi1, sem_d)
        g_u0.wait()
        g_i0.wait()
        for i in range(chunk // _LANES):
            v = pl.ds(i * _LANES, _LANES)
            out0[v] = bu0[v] + bi0[v] + _MU
        wb0 = pltpu.async_copy(out0, out_hbm.at[sl0], sem_a)
        g_u1.wait()
        g_i1.wait()
        for i in range(chunk // _LANES):
            v = pl.ds(i * _LANES, _LANES)
            out1[v] = bu1[v] + bi1[v] + _MU
        wb1 = pltpu.async_copy(out1, out_hbm.at[sl1], sem_b)
        wb0.wait()
        wb1.wait()

    return sc_kernel(
        user.astype(jnp.int32),
        item.astype(jnp.int32),
        user_biases.reshape(-1),
        item_biases.reshape(-1),
    )


# speedup vs baseline: 1.0071x; 1.0071x over previous
import functools

import jax
import jax.numpy as jnp
from jax import lax
from jax.experimental import pallas as pl
from jax.experimental.pallas import tpu as pltpu
from jax.experimental.pallas import tpu_sc as plsc

_MU = 3.5
_LANES = 16
_NCHUNK = 2


@jax.jit
def kernel(user, item, user_biases, item_biases):
    batch = user.shape[0]
    info = plsc.get_sparse_core_info()
    num_subcores = info.num_subcores
    num_cores = 1
    num_workers = num_cores * num_subcores
    b_per_w = batch // num_workers
    chunk = b_per_w // _NCHUNK

    mesh = plsc.VectorSubcoreMesh(core_axis_name="c", subcore_axis_name="s",
                                  num_cores=num_cores)

    idx_t = pltpu.VMEM((chunk,), jnp.int32)
    val_t = pltpu.VMEM((chunk,), jnp.float32)

    @functools.partial(
        pl.kernel,
        mesh=mesh,
        out_type=jax.ShapeDtypeStruct((batch,), jnp.float32),
        scratch_types=[
            idx_t, idx_t, idx_t, idx_t,
            val_t, val_t, val_t, val_t,
            val_t, val_t,
            pltpu.SemaphoreType.DMA, pltpu.SemaphoreType.DMA,
            pltpu.SemaphoreType.DMA, pltpu.SemaphoreType.DMA,
        ],
    )
    def sc_kernel(user_hbm, item_hbm, ub_hbm, ib_hbm, out_hbm,
                  uidx0, iidx0, uidx1, iidx1,
                  bu0, bi0, bu1, bi1, out0, out1,
                  sem_a, sem_b, sem_c, sem_d):
        wid = lax.axis_index("s") * num_cores + lax.axis_index("c")
        base = wid * b_per_w
        sl0 = pl.ds(base, chunk)
        sl1 = pl.ds(base + chunk, chunk)
        cp_u0 = pltpu.async_copy(user_hbm.at[sl0], uidx0, sem_a)
        cp_i0 = pltpu.async_copy(item_hbm.at[sl0], iidx0, sem_b)
        cp_u1 = pltpu.async_copy(user_hbm.at[sl1], uidx1, sem_c)
        cp_i1 = pltpu.async_copy(item_hbm.at[sl1], iidx1, sem_d)
        cp_u0.wait()
        g_u0 = pltpu.async_copy(ub_hbm.at[uidx0], bu0, sem_a)
        cp_i0.wait()
        g_i0 = pltpu.async_copy(ib_hbm.at[iidx0], bi0, sem_b)
        cp_u1.wait()
        g_u1 = pltpu.async_copy(ub_hbm.at[uidx1], bu1, sem_c)
        cp_i1.wait()
        g_i1 = pltpu.async_copy(ib_hbm.at[iidx1], bi1, sem_d)
        g_u0.wait()
        g_i0.wait()
        for i in range(chunk // _LANES):
            v = pl.ds(i * _LANES, _LANES)
            out0[v] = bu0[v] + bi0[v] + _MU
        wb0 = pltpu.async_copy(out0, out_hbm.at[sl0], sem_a)
        g_u1.wait()
        g_i1.wait()
        for i in range(chunk // _LANES):
            v = pl.ds(i * _LANES, _LANES)
            out1[v] = bu1[v] + bi1[v] + _MU
        wb1 = pltpu.async_copy(out1, out_hbm.at[sl1], sem_b)
        wb0.wait()
        wb1.wait()

    return sc_kernel(
        user.astype(jnp.int32),
        item.astype(jnp.int32),
        user_biases.reshape(-1),
        item_biases.reshape(-1),
    )
